# baseline (speedup 1.0000x reference)
import functools

import jax
import jax.numpy as jnp
from jax import lax
from jax.experimental import pallas as pl
from jax.experimental.pallas import tpu as pltpu
from jax.experimental.pallas import tpu_sc as plsc

N = 10000
NPAD = 10240
E = 320000
NC, NS = 2, 16
NW = NC * NS
EPT = E // NW
C = 500
NCH = EPT // C
ROWS_PT = NPAD // NS

_mesh = plsc.VectorSubcoreMesh(core_axis_name="c", subcore_axis_name="s")
_sc_params = pltpu.CompilerParams(use_tc_tiling_on_sc=False)


def _hist_sc(cols2d, zeros16, ones_pay):
    @functools.partial(
        pl.kernel,
        out_type=jax.ShapeDtypeStruct((NC, NPAD, 8), jnp.float32),
        mesh=_mesh,
        compiler_params=_sc_params,
        scratch_types=[
            pltpu.VMEM_SHARED((NPAD, 16), jnp.float32),
            pltpu.VMEM((NCH, C), jnp.int32),
            pltpu.VMEM((C, 16), jnp.float32),
            pltpu.SemaphoreType.DMA,
        ],
    )
    def k(cols_hbm, zeros_hbm, ones_hbm, out_hbm, acc_sh, cv, onesv, sem):
        cid = lax.axis_index("c")
        sid = lax.axis_index("s")
        wid = sid * NC + cid
        rbase = sid * ROWS_PT
        pltpu.sync_copy(ones_hbm, onesv)
        pltpu.sync_copy(cols_hbm.at[pl.ds(wid * NCH, NCH)], cv)
        pltpu.sync_copy(zeros_hbm.at[pl.ds(rbase, ROWS_PT)],
                        acc_sh.at[pl.ds(rbase, ROWS_PT)])
        plsc.subcore_barrier()

        @pl.loop(0, NCH)
        def _(j):
            pltpu.async_copy(onesv, acc_sh.at[cv.at[j]], sem, add=True)

        @pl.loop(0, NCH)
        def _(j):
            pltpu.make_async_copy(onesv, acc_sh.at[cv.at[j]], sem).wait()

        plsc.subcore_barrier()
        pltpu.sync_copy(acc_sh.at[pl.ds(rbase, ROWS_PT), pl.ds(0, 8)],
                        out_hbm.at[cid, pl.ds(rbase, ROWS_PT)])

    return k(cols2d, zeros16, ones_pay)


def _agg_sc(payload, rows2d, cols2d, width, drain_lane0=False):
    @functools.partial(
        pl.kernel,
        out_type=jax.ShapeDtypeStruct(
            (NC, NPAD, 8 if drain_lane0 else width), jnp.float32),
        mesh=_mesh,
        compiler_params=_sc_params,
        scratch_types=[
            pltpu.VMEM_SHARED((NPAD, width), jnp.float32),
            pltpu.VMEM((NCH, C), jnp.int32),
            pltpu.VMEM((NCH, C), jnp.int32),
            pltpu.VMEM((C, width), jnp.float32),
            pltpu.VMEM((C, width), jnp.float32),
            pltpu.SemaphoreType.DMA,
            pltpu.SemaphoreType.DMA,
            pltpu.SemaphoreType.DMA,
            pltpu.SemaphoreType.DMA,
        ],
    )
    def k(pay_hbm, rows_hbm, cols_hbm, out_hbm, acc_sh, rv, cv, b0, b1,
          gs0, gs1, ss0, ss1):
        cid = lax.axis_index("c")
        sid = lax.axis_index("s")
        wid = sid * NC + cid
        rbase = sid * ROWS_PT
        cbase = wid * NCH
        pltpu.sync_copy(rows_hbm.at[pl.ds(cbase, NCH)], rv)
        pltpu.sync_copy(cols_hbm.at[pl.ds(cbase, NCH)], cv)
        pltpu.async_copy(pay_hbm.at[rv.at[0]], b0, gs0)
        pltpu.sync_copy(pay_hbm.at[pl.ds(rbase, ROWS_PT)],
                        acc_sh.at[pl.ds(rbase, ROWS_PT)])
        plsc.subcore_barrier()

        @pl.loop(0, NCH, step=2)
        def _(jj):
            g1 = pltpu.async_copy(pay_hbm.at[rv.at[jj + 1]], b1, gs1)
            pltpu.make_async_copy(pay_hbm.at[rv.at[jj]], b0, gs0).wait()
            s0 = pltpu.async_copy(b0, acc_sh.at[cv.at[jj]], ss0, add=True)
            g1.wait()
            s1 = pltpu.async_copy(b1, acc_sh.at[cv.at[jj + 1]], ss1, add=True)
            s0.wait()

            @pl.when(jj + 2 < NCH)
            def _():
                pltpu.async_copy(pay_hbm.at[rv.at[jj + 2]], b0, gs0)

            s1.wait()

        plsc.subcore_barrier()
        if drain_lane0:
            pltpu.sync_copy(acc_sh.at[pl.ds(rbase, ROWS_PT), pl.ds(0, 8)],
                            out_hbm.at[cid, pl.ds(rbase, ROWS_PT)])
        else:
            pltpu.sync_copy(acc_sh.at[pl.ds(rbase, ROWS_PT)],
                            out_hbm.at[cid, pl.ds(rbase, ROWS_PT)])

    return k(payload, rows2d, cols2d)


def _dis(degp_ref):
    deg = degp_ref[0, :, 0:1] + degp_ref[1, :, 0:1] + 1.0
    return lax.rsqrt(deg)


def _mm_tc(x_pad, W1):
    def body(x_ref, w_ref, h_ref):
        h_ref[...] = jnp.dot(x_ref[...], w_ref[...],
                             preferred_element_type=jnp.float32)

    return pl.pallas_call(
        body,
        out_shape=jax.ShapeDtypeStruct((NPAD, 64), jnp.float32),
    )(x_pad, W1)


def _scale_tc(degp, h):
    def body(degp_ref, h_ref, hs_ref):
        hs_ref[...] = h_ref[...] * _dis(degp_ref)

    return pl.pallas_call(
        body,
        out_shape=jax.ShapeDtypeStruct((NPAD, 64), jnp.float32),
    )(degp, h)


def _mid_tc(p, degp, hs, b1, W2):
    def body(p_ref, degp_ref, hs_ref, b1_ref, w2_ref, zp_ref):
        dis = _dis(degp_ref)
        pre = dis * (p_ref[0] + p_ref[1] - hs_ref[...]) + b1_ref[...]
        y = jnp.maximum(pre, 0.0)
        z = jnp.dot(y, w2_ref[...], preferred_element_type=jnp.float32)
        zs = dis * z
        zp_ref[...] = lax.broadcast_in_dim(zs, (NPAD, 16), (0, 1))

    return pl.pallas_call(
        body,
        out_shape=jax.ShapeDtypeStruct((NPAD, 16), jnp.float32),
    )(p, degp, hs, b1, W2)


def _out_tc(q, degp, zp, b2):
    def body(q_ref, degp_ref, zp_ref, b2_ref, o_ref):
        dis = _dis(degp_ref)
        seg = q_ref[0, :, 0:1] + q_ref[1, :, 0:1] - zp_ref[:, 0:1]
        o_ref[...] = dis * seg + b2_ref[...]

    return pl.pallas_call(
        body,
        out_shape=jax.ShapeDtypeStruct((NPAD, 1), jnp.float32),
    )(q, degp, zp, b2)


@jax.jit
def kernel(x, edge_index, W1, b1, W2, b2):
    ei = edge_index.astype(jnp.int32)
    rows2d = ei[0].reshape(E // C, C)
    cols2d = ei[1].reshape(E // C, C)
    x_pad = jnp.pad(x, ((0, NPAD - N), (0, 0)))
    zeros16 = jnp.zeros((NPAD, 16), jnp.float32)
    ones_pay = jnp.ones((C, 16), jnp.float32)

    h = _mm_tc(x_pad, W1)
    degp = _hist_sc(cols2d, zeros16, ones_pay)
    hs = _scale_tc(degp, h)
    p = _agg_sc(hs, rows2d, cols2d, 64)
    zp = _mid_tc(p, degp, hs, b1.reshape(1, 64), W2)
    q = _agg_sc(zp, rows2d, cols2d, 16, drain_lane0=True)
    out = _out_tc(q, degp, zp, b2.reshape(1, 1))
    return out[:N]


# speedup vs baseline: 1.0382x; 1.0382x over previous
import functools

import jax
import jax.numpy as jnp
from jax import lax
from jax.experimental import pallas as pl
from jax.experimental.pallas import tpu as pltpu
from jax.experimental.pallas import tpu_sc as plsc

N = 10000
NPAD = 10240
E = 320000
NC, NS = 2, 16
NW = NC * NS
EPT = E // NW
C = 500
NCH = EPT // C
ROWS_PT = NPAD // NS

_mesh = plsc.VectorSubcoreMesh(core_axis_name="c", subcore_axis_name="s")
_sc_params = pltpu.CompilerParams(use_tc_tiling_on_sc=False,
                                  disable_bounds_checks=True)


def _hist_sc(cols2d, zeros16, ones_pay):
    @functools.partial(
        pl.kernel,
        out_type=jax.ShapeDtypeStruct((NC, NPAD, 16), jnp.float32),
        mesh=_mesh,
        compiler_params=_sc_params,
        scratch_types=[
            pltpu.VMEM_SHARED((NPAD, 16), jnp.float32),
            pltpu.VMEM((NCH, C), jnp.int32),
            pltpu.VMEM((C, 16), jnp.float32),
            pltpu.SemaphoreType.DMA,
        ],
    )
    def k(cols_hbm, zeros_hbm, ones_hbm, out_hbm, acc_sh, cv, onesv, sem):
        cid = lax.axis_index("c")
        sid = lax.axis_index("s")
        wid = sid * NC + cid
        rbase = sid * ROWS_PT
        pltpu.sync_copy(ones_hbm, onesv)
        pltpu.sync_copy(cols_hbm.at[pl.ds(wid * NCH, NCH)], cv)
        pltpu.sync_copy(zeros_hbm.at[pl.ds(rbase, ROWS_PT)],
                        acc_sh.at[pl.ds(rbase, ROWS_PT)])
        plsc.subcore_barrier()

        @pl.loop(0, NCH)
        def _(j):
            pltpu.async_copy(onesv, acc_sh.at[cv.at[j]], sem, add=True)

        @pl.loop(0, NCH)
        def _(j):
            pltpu.make_async_copy(onesv, acc_sh.at[cv.at[j]], sem).wait()

        plsc.subcore_barrier()
        pltpu.sync_copy(acc_sh.at[pl.ds(rbase, ROWS_PT)],
                        out_hbm.at[cid, pl.ds(rbase, ROWS_PT)])

    return k(cols2d, zeros16, ones_pay)


def _agg_sc(payload, rows2d, cols2d, width):
    @functools.partial(
        pl.kernel,
        out_type=jax.ShapeDtypeStruct((NC, NPAD, width), jnp.float32),
        mesh=_mesh,
        compiler_params=_sc_params,
        scratch_types=[
            pltpu.VMEM_SHARED((NPAD, width), jnp.float32),
            pltpu.VMEM((NCH, C), jnp.int32),
            pltpu.VMEM((NCH, C), jnp.int32),
            pltpu.VMEM((C, width), jnp.float32),
            pltpu.VMEM((C, width), jnp.float32),
            pltpu.SemaphoreType.DMA,
            pltpu.SemaphoreType.DMA,
            pltpu.SemaphoreType.DMA,
            pltpu.SemaphoreType.DMA,
        ],
    )
    def k(pay_hbm, rows_hbm, cols_hbm, out_hbm, acc_sh, rv, cv, b0, b1,
          gs0, gs1, ss0, ss1):
        cid = lax.axis_index("c")
        sid = lax.axis_index("s")
        wid = sid * NC + cid
        rbase = sid * ROWS_PT
        cbase = wid * NCH
        pltpu.sync_copy(rows_hbm.at[pl.ds(cbase, NCH)], rv)
        pltpu.sync_copy(cols_hbm.at[pl.ds(cbase, NCH)], cv)
        pltpu.async_copy(pay_hbm.at[rv.at[0]], b0, gs0)
        pltpu.sync_copy(pay_hbm.at[pl.ds(rbase, ROWS_PT)],
                        acc_sh.at[pl.ds(rbase, ROWS_PT)])
        plsc.subcore_barrier()

        @pl.loop(0, NCH, step=2)
        def _(jj):
            g1 = pltpu.async_copy(pay_hbm.at[rv.at[jj + 1]], b1, gs1)
            pltpu.make_async_copy(pay_hbm.at[rv.at[jj]], b0, gs0).wait()
            s0 = pltpu.async_copy(b0, acc_sh.at[cv.at[jj]], ss0, add=True)
            g1.wait()
            s1 = pltpu.async_copy(b1, acc_sh.at[cv.at[jj + 1]], ss1, add=True)
            s0.wait()

            @pl.when(jj + 2 < NCH)
            def _():
                pltpu.async_copy(pay_hbm.at[rv.at[jj + 2]], b0, gs0)

            s1.wait()

        plsc.subcore_barrier()
        pltpu.sync_copy(acc_sh.at[pl.ds(rbase, ROWS_PT)],
                        out_hbm.at[cid, pl.ds(rbase, ROWS_PT)])

    return k(payload, rows2d, cols2d)


def _dis(degp_ref):
    deg = degp_ref[0, :, 0:1] + degp_ref[1, :, 0:1] + 1.0
    return lax.rsqrt(deg)


def _mm_tc(x_pad, W1):
    def body(x_ref, w_ref, h_ref):
        h_ref[...] = jnp.dot(x_ref[...], w_ref[...],
                             preferred_element_type=jnp.float32)

    return pl.pallas_call(
        body,
        out_shape=jax.ShapeDtypeStruct((NPAD, 64), jnp.float32),
    )(x_pad, W1)


def _scale_tc(degp, h):
    def body(degp_ref, h_ref, hs_ref):
        hs_ref[...] = h_ref[...] * _dis(degp_ref)

    return pl.pallas_call(
        body,
        out_shape=jax.ShapeDtypeStruct((NPAD, 64), jnp.float32),
    )(degp, h)


def _mid_tc(p, degp, hs, b1, W2):
    def body(p_ref, degp_ref, hs_ref, b1_ref, w2_ref, zp_ref):
        dis = _dis(degp_ref)
        pre = dis * (p_ref[0] + p_ref[1] - hs_ref[...]) + b1_ref[...]
        y = jnp.maximum(pre, 0.0)
        z = jnp.dot(y, w2_ref[...], preferred_element_type=jnp.float32)
        zs = dis * z
        zp_ref[...] = lax.broadcast_in_dim(zs, (NPAD, 16), (0, 1))

    return pl.pallas_call(
        body,
        out_shape=jax.ShapeDtypeStruct((NPAD, 16), jnp.float32),
    )(p, degp, hs, b1, W2)


def _out_tc(q, degp, zp, b2):
    def body(q_ref, degp_ref, zp_ref, b2_ref, o_ref):
        dis = _dis(degp_ref)
        seg = q_ref[0, :, 0:1] + q_ref[1, :, 0:1] - zp_ref[:, 0:1]
        o_ref[...] = dis * seg + b2_ref[...]

    return pl.pallas_call(
        body,
        out_shape=jax.ShapeDtypeStruct((NPAD, 1), jnp.float32),
    )(q, degp, zp, b2)


@jax.jit
def kernel(x, edge_index, W1, b1, W2, b2):
    ei = edge_index.astype(jnp.int32)
    rows2d = ei[0].reshape(E // C, C)
    cols2d = ei[1].reshape(E // C, C)
    x_pad = jnp.pad(x, ((0, NPAD - N), (0, 0)))
    zeros16 = jnp.zeros((NPAD, 16), jnp.float32)
    ones_pay = jnp.ones((C, 16), jnp.float32)

    h = _mm_tc(x_pad, W1)
    degp = _hist_sc(cols2d, zeros16, ones_pay)
    hs = _scale_tc(degp, h)
    p = _agg_sc(hs, rows2d, cols2d, 64)
    zp = _mid_tc(p, degp, hs, b1.reshape(1, 64), W2)
    q = _agg_sc(zp, rows2d, cols2d, 16)
    out = _out_tc(q, degp, zp, b2.reshape(1, 1))
    return out[:N]


# speedup vs baseline: 1.0532x; 1.0144x over previous
import functools

import jax
import jax.numpy as jnp
from jax import lax
from jax.experimental import pallas as pl
from jax.experimental.pallas import tpu as pltpu
from jax.experimental.pallas import tpu_sc as plsc

N = 10000
NPAD = 10240
E = 320000
NC, NS = 2, 16
NW = NC * NS
EPT = E // NW
C = 500
NCH = EPT // C
C16 = 1000
NCH16 = EPT // C16
ROWS_PT = NPAD // NS

_mesh = plsc.VectorSubcoreMesh(core_axis_name="c", subcore_axis_name="s")
_sc_params = pltpu.CompilerParams(use_tc_tiling_on_sc=False)


def _hist_sc(cols2d, zeros16, ones_pay):
    @functools.partial(
        pl.kernel,
        out_type=jax.ShapeDtypeStruct((NC, NPAD, 16), jnp.float32),
        mesh=_mesh,
        compiler_params=_sc_params,
        scratch_types=[
            pltpu.VMEM_SHARED((NPAD, 16), jnp.float32),
            pltpu.VMEM((NCH16, C16), jnp.int32),
            pltpu.VMEM((C16, 16), jnp.float32),
            pltpu.SemaphoreType.DMA,
        ],
    )
    def k(cols_hbm, zeros_hbm, ones_hbm, out_hbm, acc_sh, cv, onesv, sem):
        cid = lax.axis_index("c")
        sid = lax.axis_index("s")
        wid = sid * NC + cid
        rbase = sid * ROWS_PT
        pltpu.sync_copy(ones_hbm, onesv)
        pltpu.sync_copy(cols_hbm.at[pl.ds(wid * NCH16, NCH16)], cv)
        pltpu.sync_copy(zeros_hbm.at[pl.ds(rbase, ROWS_PT)],
                        acc_sh.at[pl.ds(rbase, ROWS_PT)])
        plsc.subcore_barrier()

        @pl.loop(0, NCH16)
        def _(j):
            pltpu.async_copy(onesv, acc_sh.at[cv.at[j]], sem, add=True)

        @pl.loop(0, NCH16)
        def _(j):
            pltpu.make_async_copy(onesv, acc_sh.at[cv.at[j]], sem).wait()

        plsc.subcore_barrier()
        pltpu.sync_copy(acc_sh.at[pl.ds(rbase, ROWS_PT)],
                        out_hbm.at[cid, pl.ds(rbase, ROWS_PT)])

    return k(cols2d, zeros16, ones_pay)


def _agg_sc(payload, rows2d, cols2d, width, c, nch):
    @functools.partial(
        pl.kernel,
        out_type=jax.ShapeDtypeStruct((NC, NPAD, width), jnp.float32),
        mesh=_mesh,
        compiler_params=_sc_params,
        scratch_types=[
            pltpu.VMEM_SHARED((NPAD, width), jnp.float32),
            pltpu.VMEM((nch, c), jnp.int32),
            pltpu.VMEM((nch, c), jnp.int32),
            pltpu.VMEM((c, width), jnp.float32),
            pltpu.VMEM((c, width), jnp.float32),
            pltpu.SemaphoreType.DMA,
            pltpu.SemaphoreType.DMA,
            pltpu.SemaphoreType.DMA,
            pltpu.SemaphoreType.DMA,
        ],
    )
    def k(pay_hbm, rows_hbm, cols_hbm, out_hbm, acc_sh, rv, cv, b0, b1,
          gs0, gs1, ss0, ss1):
        cid = lax.axis_index("c")
        sid = lax.axis_index("s")
        wid = sid * NC + cid
        rbase = sid * ROWS_PT
        cbase = wid * nch
        pltpu.sync_copy(rows_hbm.at[pl.ds(cbase, nch)], rv)
        pltpu.sync_copy(cols_hbm.at[pl.ds(cbase, nch)], cv)
        pltpu.async_copy(pay_hbm.at[rv.at[0]], b0, gs0)
        pltpu.sync_copy(pay_hbm.at[pl.ds(rbase, ROWS_PT)],
                        acc_sh.at[pl.ds(rbase, ROWS_PT)])
        plsc.subcore_barrier()

        @pl.loop(0, nch, step=2)
        def _(jj):
            g1 = pltpu.async_copy(pay_hbm.at[rv.at[jj + 1]], b1, gs1)
            pltpu.make_async_copy(pay_hbm.at[rv.at[jj]], b0, gs0).wait()
            s0 = pltpu.async_copy(b0, acc_sh.at[cv.at[jj]], ss0, add=True)
            g1.wait()
            s1 = pltpu.async_copy(b1, acc_sh.at[cv.at[jj + 1]], ss1, add=True)
            s0.wait()

            @pl.when(jj + 2 < nch)
            def _():
                pltpu.async_copy(pay_hbm.at[rv.at[jj + 2]], b0, gs0)

            s1.wait()

        plsc.subcore_barrier()
        pltpu.sync_copy(acc_sh.at[pl.ds(rbase, ROWS_PT)],
                        out_hbm.at[cid, pl.ds(rbase, ROWS_PT)])

    return k(payload, rows2d, cols2d)


def _dis(degp_ref):
    deg = degp_ref[0, :, 0:1] + degp_ref[1, :, 0:1] + 1.0
    return lax.rsqrt(deg)


def _mm_tc(x_pad, W1):
    def body(x_ref, w_ref, h_ref):
        h_ref[...] = jnp.dot(x_ref[...], w_ref[...],
                             preferred_element_type=jnp.float32)

    return pl.pallas_call(
        body,
        out_shape=jax.ShapeDtypeStruct((NPAD, 64), jnp.float32),
    )(x_pad, W1)


def _scale_tc(degp, h):
    def body(degp_ref, h_ref, hs_ref):
        hs_ref[...] = h_ref[...] * _dis(degp_ref)

    return pl.pallas_call(
        body,
        out_shape=jax.ShapeDtypeStruct((NPAD, 64), jnp.float32),
    )(degp, h)


def _mid_tc(p, degp, hs, b1, W2):
    def body(p_ref, degp_ref, hs_ref, b1_ref, w2_ref, zp_ref):
        dis = _dis(degp_ref)
        pre = dis * (p_ref[0] + p_ref[1] - hs_ref[...]) + b1_ref[...]
        y = jnp.maximum(pre, 0.0)
        z = jnp.dot(y, w2_ref[...], preferred_element_type=jnp.float32)
        zs = dis * z
        zp_ref[...] = lax.broadcast_in_dim(zs, (NPAD, 16), (0, 1))

    return pl.pallas_call(
        body,
        out_shape=jax.ShapeDtypeStruct((NPAD, 16), jnp.float32),
    )(p, degp, hs, b1, W2)


def _out_tc(q, degp, zp, b2):
    def body(q_ref, degp_ref, zp_ref, b2_ref, o_ref):
        dis = _dis(degp_ref)
        seg = q_ref[0, :, 0:1] + q_ref[1, :, 0:1] - zp_ref[:, 0:1]
        o_ref[...] = dis * seg + b2_ref[...]

    return pl.pallas_call(
        body,
        out_shape=jax.ShapeDtypeStruct((NPAD, 1), jnp.float32),
    )(q, degp, zp, b2)


@jax.jit
def kernel(x, edge_index, W1, b1, W2, b2):
    ei = edge_index.astype(jnp.int32)
    rows2d = ei[0].reshape(E // C, C)
    cols2d = ei[1].reshape(E // C, C)
    rows2d16 = ei[0].reshape(E // C16, C16)
    cols2d16 = ei[1].reshape(E // C16, C16)
    x_pad = jnp.pad(x, ((0, NPAD - N), (0, 0)))
    zeros16 = jnp.zeros((NPAD, 16), jnp.float32)
    ones_pay = jnp.ones((C16, 16), jnp.float32)

    h = _mm_tc(x_pad, W1)
    degp = _hist_sc(cols2d16, zeros16, ones_pay)
    hs = _scale_tc(degp, h)
    p = _agg_sc(hs, rows2d, cols2d, 64, C, NCH)
    zp = _mid_tc(p, degp, hs, b1.reshape(1, 64), W2)
    q = _agg_sc(zp, rows2d16, cols2d16, 16, C16, NCH16)
    out = _out_tc(q, degp, zp, b2.reshape(1, 1))
    return out[:N]
